# baseline (speedup 1.0000x reference)
import functools

import jax
import jax.numpy as jnp
from jax import lax
from jax.experimental import pallas as pl
from jax.experimental.pallas import tpu as pltpu
from jax.experimental.pallas import tpu_sc as plsc

TYPE_NUM = 100
DIM = 128
N_NODES = 100000
NC = 2
NS = 16
NW = NC * NS

BIGROW = 320
GPARTS = (128, 128, 64)
NBUF = 2
N_ROUNDS = 5
SPAN = BIGROW * NBUF * N_ROUNDS
BASE_UNITS = (N_NODES - SPAN) // 8

_mesh = plsc.VectorSubcoreMesh(core_axis_name="c", subcore_axis_name="s")


@functools.partial(
    pl.kernel,
    out_type=jax.ShapeDtypeStruct((N_NODES, DIM), jnp.float32),
    mesh=_mesh,
    scratch_types=(
        [pltpu.VMEM((SPAN,), jnp.int32),
         pltpu.VMEM_SHARED((TYPE_NUM, DIM), jnp.float32)]
        + [pltpu.VMEM((BIGROW, DIM), jnp.float32) for _ in range(NBUF)]
        + [pltpu.SemaphoreType.DMA for _ in range(2 * NBUF)]
    ),
)
def _embed(idx_hbm, table_hbm, out_hbm, idx_v, table_sp, *bufs_and_sems):
    rows = bufs_and_sems[:NBUF]
    gsem = bufs_and_sems[NBUF:2 * NBUF]
    osem = bufs_and_sems[2 * NBUF:]

    sid = lax.axis_index("s")
    wid = sid * NC + lax.axis_index("c")
    base = pl.multiple_of((wid * BASE_UNITS) // (NW - 1) * 8, 8)

    @pl.when(sid == 0)
    def _():
        pltpu.sync_copy(table_hbm, table_sp)

    pltpu.sync_copy(idx_hbm.at[pl.ds(base, SPAN)], idx_v)
    plsc.subcore_barrier()

    def gather_descrs(b, loff):
        ds = []
        part_off = 0
        for size in GPARTS:
            ds.append(pltpu.make_async_copy(
                table_sp.at[idx_v.at[pl.ds(loff + part_off, size)]],
                rows[b].at[pl.ds(part_off, size)], gsem[b]))
            part_off += size
        return ds

    def out_drain(b):
        pltpu.make_async_copy(rows[b], out_hbm.at[pl.ds(0, BIGROW)],
                              osem[b]).wait()

    for b in range(NBUF):
        for d in gather_descrs(b, b * BIGROW):
            d.start()

    def round_body(g, carry):
        @pl.when(g > 0)
        def _():
            for b in range(NBUF):
                loff = pl.multiple_of((g * NBUF + b) * BIGROW, 8)
                out_drain(b)
                for d in gather_descrs(b, loff):
                    d.start()

        for b in range(NBUF):
            loff = pl.multiple_of((g * NBUF + b) * BIGROW, 8)
            for d in gather_descrs(b, loff):
                d.wait()
            pltpu.make_async_copy(rows[b],
                                  out_hbm.at[pl.ds(base + loff, BIGROW)],
                                  osem[b]).start()
        return carry

    lax.fori_loop(0, N_ROUNDS, round_body, 0)

    for b in range(NBUF):
        out_drain(b)


def kernel(node_type, table):
    return _embed(node_type.astype(jnp.int32), table)


# speedup vs baseline: 1.2295x; 1.2295x over previous
import functools

import jax
import jax.numpy as jnp
from jax import lax
from jax.experimental import pallas as pl
from jax.experimental.pallas import tpu as pltpu
from jax.experimental.pallas import tpu_sc as plsc

TYPE_NUM = 100
DIM = 128
N_NODES = 100000
NC = 2
NS = 16
NW = NC * NS

CHUNK = 128
NBUF = 5
N_ROUNDS = 5
SPAN = CHUNK * NBUF * N_ROUNDS
BASE_UNITS = (N_NODES - SPAN) // 8

_mesh = plsc.VectorSubcoreMesh(core_axis_name="c", subcore_axis_name="s")


@functools.partial(
    pl.kernel,
    out_type=jax.ShapeDtypeStruct((N_NODES, DIM), jnp.float32),
    mesh=_mesh,
    scratch_types=(
        [pltpu.VMEM((SPAN,), jnp.int32),
         pltpu.VMEM_SHARED((TYPE_NUM, DIM), jnp.float32)]
        + [pltpu.VMEM((CHUNK, DIM), jnp.float32) for _ in range(NBUF)]
        + [pltpu.SemaphoreType.DMA for _ in range(2 * NBUF)]
    ),
)
def _embed(idx_hbm, table_hbm, out_hbm, idx_v, table_sp, *bufs_and_sems):
    rows = bufs_and_sems[:NBUF]
    gsem = bufs_and_sems[NBUF:2 * NBUF]
    osem = bufs_and_sems[2 * NBUF:]

    sid = lax.axis_index("s")
    wid = sid * NC + lax.axis_index("c")
    base = pl.multiple_of((wid * BASE_UNITS) // (NW - 1) * 8, 8)

    @pl.when(sid == 0)
    def _():
        pltpu.sync_copy(table_hbm, table_sp)

    pltpu.sync_copy(idx_hbm.at[pl.ds(base, SPAN)], idx_v)
    plsc.subcore_barrier()

    def gather_descr(b, loff):
        return pltpu.make_async_copy(
            table_sp.at[idx_v.at[pl.ds(loff, CHUNK)]], rows[b], gsem[b])

    def out_drain(b):
        pltpu.make_async_copy(rows[b], out_hbm.at[pl.ds(0, CHUNK)],
                              osem[b]).wait()

    for b in range(NBUF):
        gather_descr(b, b * CHUNK).start()

    def round_body(g, carry):
        @pl.when(g > 0)
        def _():
            for b in range(NBUF):
                loff = pl.multiple_of((g * NBUF + b) * CHUNK, 8)
                out_drain(b)
                gather_descr(b, loff).start()

        for b in range(NBUF):
            loff = pl.multiple_of((g * NBUF + b) * CHUNK, 8)
            gather_descr(b, loff).wait()
            pltpu.make_async_copy(rows[b], out_hbm.at[pl.ds(base + loff, CHUNK)],
                                  osem[b]).start()
        return carry

    lax.fori_loop(0, N_ROUNDS, round_body, 0)

    for b in range(NBUF):
        out_drain(b)


def kernel(node_type, table):
    return _embed(node_type.astype(jnp.int32), table)


# speedup vs baseline: 1.2418x; 1.0100x over previous
import functools

import jax
import jax.numpy as jnp
from jax import lax
from jax.experimental import pallas as pl
from jax.experimental.pallas import tpu as pltpu
from jax.experimental.pallas import tpu_sc as plsc

TYPE_NUM = 100
DIM = 128
N_NODES = 100000
NC = 2
NS = 16
NW = NC * NS

CHUNK = 64
NBUF = 10
N_ROUNDS = 5
SPAN = CHUNK * NBUF * N_ROUNDS
BASE_UNITS = (N_NODES - SPAN) // 8

_mesh = plsc.VectorSubcoreMesh(core_axis_name="c", subcore_axis_name="s")


@functools.partial(
    pl.kernel,
    out_type=jax.ShapeDtypeStruct((N_NODES, DIM), jnp.float32),
    mesh=_mesh,
    scratch_types=(
        [pltpu.VMEM((SPAN,), jnp.int32),
         pltpu.VMEM_SHARED((TYPE_NUM, DIM), jnp.float32)]
        + [pltpu.VMEM((CHUNK, DIM), jnp.float32) for _ in range(NBUF)]
        + [pltpu.SemaphoreType.DMA for _ in range(2 * NBUF)]
    ),
)
def _embed(idx_hbm, table_hbm, out_hbm, idx_v, table_sp, *bufs_and_sems):
    rows = bufs_and_sems[:NBUF]
    gsem = bufs_and_sems[NBUF:2 * NBUF]
    osem = bufs_and_sems[2 * NBUF:]

    sid = lax.axis_index("s")
    wid = sid * NC + lax.axis_index("c")
    base = pl.multiple_of((wid * BASE_UNITS) // (NW - 1) * 8, 8)

    @pl.when(sid == 0)
    def _():
        pltpu.sync_copy(table_hbm, table_sp)

    pltpu.sync_copy(idx_hbm.at[pl.ds(base, SPAN)], idx_v)
    plsc.subcore_barrier()

    def gather_descr(b, loff):
        return pltpu.make_async_copy(
            table_sp.at[idx_v.at[pl.ds(loff, CHUNK)]], rows[b], gsem[b])

    def out_drain(b):
        pltpu.make_async_copy(rows[b], out_hbm.at[pl.ds(0, CHUNK)],
                              osem[b]).wait()

    for b in range(NBUF):
        gather_descr(b, b * CHUNK).start()

    def round_body(g, carry):
        @pl.when(g > 0)
        def _():
            for b in range(NBUF):
                loff = pl.multiple_of((g * NBUF + b) * CHUNK, 8)
                out_drain(b)
                gather_descr(b, loff).start()

        for b in range(NBUF):
            loff = pl.multiple_of((g * NBUF + b) * CHUNK, 8)
            gather_descr(b, loff).wait()
            pltpu.make_async_copy(rows[b], out_hbm.at[pl.ds(base + loff, CHUNK)],
                                  osem[b]).start()
        return carry

    lax.fori_loop(0, N_ROUNDS, round_body, 0)

    for b in range(NBUF):
        out_drain(b)


def kernel(node_type, table):
    return _embed(node_type.astype(jnp.int32), table)
